# Initial kernel scaffold; baseline (speedup 1.0000x reference)
import jax
import jax.numpy as jnp
from jax.experimental import pallas as pl


def kernel(outputs, labels):
    raise NotImplementedError("write your pallas kernel here")


# speedup vs baseline: 3.5079x; 3.5079x over previous
import functools

import jax
import jax.numpy as jnp
from jax import lax
from jax.experimental import pallas as pl
from jax.experimental.pallas import tpu as pltpu
from jax.experimental.pallas import tpu_sc as plsc

B = 128
N = 8192
M = 512
K = 5
INV_TAU = 16.0
NC = 2
NS = 16
L = 16
NW = NC * NS
RPW = B // NW
CHUNKS = N // L
NEG = float("-inf")
MASK31 = 0x7FFFFFFF


def _to_key(vals):
    bits = lax.bitcast_convert_type(vals, jnp.int32)
    return jnp.where(bits >= 0, bits, bits ^ MASK31)


def _from_key(keys):
    bits = jnp.where(keys >= 0, keys, keys ^ MASK31)
    return lax.bitcast_convert_type(bits, jnp.float32)


_mesh = plsc.VectorSubcoreMesh(core_axis_name="c", subcore_axis_name="s")


@functools.partial(
    pl.kernel,
    out_type=jax.ShapeDtypeStruct((NW, L), jnp.float32),
    mesh=_mesh,
    compiler_params=pltpu.CompilerParams(needs_layout_passes=False),
    scratch_types=[
        pltpu.VMEM((N,), jnp.float32),
        pltpu.VMEM((N,), jnp.int32),
        pltpu.VMEM((B,), jnp.int32),
        pltpu.VMEM((L,), jnp.float32),
    ],
)
def _sc_loss_kernel(outputs_hbm, labels_hbm, out_hbm, row_v, keys_v,
                    labels_v, stage_v):
    wid = lax.axis_index("s") * NC + lax.axis_index("c")
    pltpu.sync_copy(labels_hbm, labels_v)
    lane = lax.iota(jnp.int32, L)
    lane_f = lane.astype(jnp.float32)
    zero_f = jnp.zeros((L,), jnp.float32)
    zero_i = jnp.zeros((L,), jnp.int32)
    one_i = jnp.ones((L,), jnp.int32)
    neg_v = jnp.full((L,), NEG, jnp.float32)

    pvec = zero_f
    for j in range(RPW):
        r = wid * RPW + j
        pltpu.sync_copy(outputs_hbm.at[r], row_v)
        lab_vec = plsc.load_gather(labels_v, [jnp.full((L,), r, jnp.int32)])

        def p1_body(i, carry):
            t_acc, max_acc = carry
            chunk = row_v[pl.ds(i * L, L)]
            keys_v[pl.ds(i * L, L)] = _to_key(chunk)
            gidx = jnp.full((L,), i * L, jnp.int32) + lane
            t_acc = jnp.maximum(t_acc, jnp.where(gidx == lab_vec, chunk, NEG))
            max_acc = jnp.maximum(max_acc, chunk)
            return t_acc, max_acc

        t_acc, max_acc = lax.fori_loop(0, CHUNKS, p1_body, (neg_v, neg_v))
        t_s = jnp.max(t_acc)
        t_vec = jnp.full((L,), t_s)
        m1_s = jnp.max(max_acc)

        levels = [jnp.full((L,), m1_s)]
        counts = []
        for _p in range(K - 1):
            prev = levels[-1]

            def lvl_body(i, carry, prev=prev):
                m_acc, c_acc = carry
                chunk = row_v[pl.ds(i * L, L)]
                m_acc = jnp.maximum(
                    m_acc, jnp.where(chunk < prev, chunk, NEG))
                c_acc = c_acc + jnp.where(chunk == prev, one_i, zero_i)
                return m_acc, c_acc

            m_acc, c_acc = lax.fori_loop(0, CHUNKS, lvl_body, (neg_v, zero_i))
            counts.append(jnp.sum(c_acc))
            levels.append(jnp.full((L,), jnp.max(m_acc)))

        last = levels[-1]

        def cnt_body(i, c_acc, last=last):
            chunk = row_v[pl.ds(i * L, L)]
            return c_acc + jnp.where(chunk == last, one_i, zero_i)

        counts.append(jnp.sum(lax.fori_loop(0, CHUNKS, cnt_body, zero_i)))

        t_key_s = jnp.max(_to_key(t_vec))

        def count_ge(mid_s):
            mid_v = jnp.full((L,), mid_s)

            def cb(i, acc):
                kk = keys_v[pl.ds(i * L, L)]
                return acc + jnp.where(kk >= mid_v, one_i, zero_i)

            acc = lax.fori_loop(0, CHUNKS, cb, zero_i)
            return jnp.sum(acc) - jnp.where(
                t_key_s >= mid_s, jnp.int32(1), jnp.int32(0))

        target = jnp.int32(M - 1)
        c0 = count_ge(jnp.int32(0))
        big = c0 >= target
        lo = jnp.where(big, jnp.int32(0), jnp.int32(-2147483648))
        hi = jnp.where(big, jnp.int32(2147483647), jnp.int32(-1))

        def bs_body(_i, carry):
            lo, hi = carry
            mid = lo + 1 + (hi - lo - 1) // 2
            good = count_ge(mid) >= target
            return jnp.where(good, mid, lo), jnp.where(good, hi, mid - 1)

        lo, hi = lax.fori_loop(0, 31, bs_body, (lo, hi))
        theta_vec = _from_key(jnp.full((L,), lo))

        w1 = levels[0]

        def fin_body(i, carry):
            sexp, z0, z1, z2, z3, z4, cnt = carry
            chunk = row_v[pl.ds(i * L, L)]
            gt = chunk > theta_vec
            cnt = cnt + jnp.where(gt, one_i, zero_i)
            sexp = sexp + jnp.where(gt, jnp.exp(chunk - w1), 0.0)
            z0 = z0 + jnp.where(
                gt, jnp.exp(-jnp.abs(chunk - levels[0]) * INV_TAU), 0.0)
            z1 = z1 + jnp.where(
                gt, jnp.exp(-jnp.abs(chunk - levels[1]) * INV_TAU), 0.0)
            z2 = z2 + jnp.where(
                gt, jnp.exp(-jnp.abs(chunk - levels[2]) * INV_TAU), 0.0)
            z3 = z3 + jnp.where(
                gt, jnp.exp(-jnp.abs(chunk - levels[3]) * INV_TAU), 0.0)
            z4 = z4 + jnp.where(
                gt, jnp.exp(-jnp.abs(chunk - levels[4]) * INV_TAU), 0.0)
            return sexp, z0, z1, z2, z3, z4, cnt

        sexp, z0, z1, z2, z3, z4, cnt = lax.fori_loop(
            0, CHUNKS, fin_body,
            (zero_f, zero_f, zero_f, zero_f, zero_f, zero_f, zero_i))
        zsums = [jnp.full((L,), jnp.sum(z)) for z in (z0, z1, z2, z3, z4)]
        sexp_v = jnp.full((L,), jnp.sum(sexp))
        cnt_v = jnp.full((L,), jnp.sum(cnt))

        tgt = t_vec > theta_vec
        cnt_false = cnt_v - jnp.where(tgt, one_i, zero_i)
        r_f = (jnp.full((L,), jnp.int32(M - 1)) - cnt_false).astype(
            jnp.float32)
        et = jnp.exp(t_vec - w1)
        sum_exp = (sexp_v + jnp.where(tgt, zero_f, et)
                   + r_f * jnp.exp(theta_vec - w1))
        pv = 0.2 * et / sum_exp

        cum = jnp.int32(0)
        for p in range(K):
            a_v = jnp.full((L,), cum)
            cum = cum + counts[p]
            b_v = jnp.full((L,), cum)
            cmask = (lane >= a_v) & (lane < b_v) & (lane < K)
            coeff_lane = 0.2 * jnp.minimum(float(K) - lane_f, 4.0)
            cs_v = jnp.full((L,), jnp.sum(jnp.where(cmask, coeff_lane, 0.0)))
            numer = jnp.exp(-jnp.abs(t_vec - levels[p]) * INV_TAU)
            z_tot = (zsums[p] + jnp.where(tgt, zero_f, numer)
                     + r_f * jnp.exp(-jnp.abs(theta_vec - levels[p])
                                     * INV_TAU))
            pv = pv + cs_v * numer / jnp.maximum(z_tot, 1e-30)

        pvec = jnp.where(lane == j, pv, pvec)

    stage_v[...] = pvec
    pltpu.sync_copy(stage_v, out_hbm.at[wid])


def kernel(outputs, labels):
    p2d = _sc_loss_kernel(outputs, labels)
    p = p2d[:, :RPW].reshape(B)
    return jnp.mean(-jnp.log(p * (1.0 - 2e-07) + 1e-07))


# speedup vs baseline: 8.7958x; 2.5075x over previous
import functools

import jax
import jax.numpy as jnp
from jax import lax
from jax.experimental import pallas as pl
from jax.experimental.pallas import tpu as pltpu
from jax.experimental.pallas import tpu_sc as plsc

B = 128
N = 8192
M = 512
K = 5
INV_TAU = 16.0
NC = 2
NS = 16
L = 16
NW = NC * NS
RPW = B // NW
CHUNKS = N // L
UNR = 4
NEG = float("-inf")
MININT = -2147483648
MASK31 = 0x7FFFFFFF

_mesh = plsc.VectorSubcoreMesh(core_axis_name="c", subcore_axis_name="s")


def _to_ukey(bits):
    return bits ^ (lax.shift_right_arithmetic(bits, 31) | MININT)


@functools.partial(
    pl.kernel,
    out_type=jax.ShapeDtypeStruct((NW, L), jnp.float32),
    mesh=_mesh,
    compiler_params=pltpu.CompilerParams(needs_layout_passes=False),
    scratch_types=[
        pltpu.VMEM((N,), jnp.float32),
        pltpu.VMEM((N,), jnp.int32),
        pltpu.VMEM((B,), jnp.int32),
        pltpu.VMEM((256,), jnp.int32),
        pltpu.VMEM((L,), jnp.float32),
    ],
)
def _sc_loss_kernel(outputs_hbm, labels_hbm, out_hbm, row_v, keys_v,
                    labels_v, hist_v, stage_v):
    wid = lax.axis_index("s") * NC + lax.axis_index("c")
    pltpu.sync_copy(labels_hbm, labels_v)
    lane = lax.iota(jnp.int32, L)
    lane_f = lane.astype(jnp.float32)
    zero_f = jnp.zeros((L,), jnp.float32)
    zero_i = jnp.zeros((L,), jnp.int32)
    one_i = jnp.ones((L,), jnp.int32)
    neg_v = jnp.full((L,), NEG, jnp.float32)

    pvec = zero_f
    for j in range(RPW):
        r = wid * RPW + j
        pltpu.sync_copy(outputs_hbm.at[r], row_v)
        lab_vec = plsc.load_gather(labels_v, [jnp.full((L,), r, jnp.int32)])
        t_vec = plsc.load_gather(row_v, [lab_vec])

        def p1_body(i, carry):
            s1, s2, s3, s4, s5 = carry
            for u in range(UNR):
                chunk = row_v[pl.ds(i * (UNR * L) + u * L, L)]
                bits = lax.bitcast_convert_type(chunk, jnp.int32)
                keys_v[pl.ds(i * (UNR * L) + u * L, L)] = _to_ukey(bits)
                x = chunk
                n1 = jnp.maximum(s1, x)
                x = jnp.minimum(s1, x)
                n2 = jnp.maximum(s2, x)
                x = jnp.minimum(s2, x)
                n3 = jnp.maximum(s3, x)
                x = jnp.minimum(s3, x)
                n4 = jnp.maximum(s4, x)
                x = jnp.minimum(s4, x)
                n5 = jnp.maximum(s5, x)
                s1, s2, s3, s4, s5 = n1, n2, n3, n4, n5
            return s1, s2, s3, s4, s5

        svecs = list(lax.fori_loop(0, CHUNKS // UNR, p1_body, (neg_v,) * 5))

        levels = [jnp.full((L,), jnp.max(svecs[0]))]
        counts = []
        for _p in range(K - 1):
            prev = levels[-1]
            macc, cacc = neg_v, zero_i
            for s in svecs:
                macc = jnp.maximum(macc, jnp.where(s < prev, s, NEG))
                cacc = cacc + jnp.where(s == prev, one_i, zero_i)
            counts.append(jnp.sum(cacc))
            levels.append(jnp.full((L,), jnp.max(macc)))
        cacc = zero_i
        for s in svecs:
            cacc = cacc + jnp.where(s == levels[-1], one_i, zero_i)
        counts.append(jnp.sum(cacc))

        t_bits = lax.bitcast_convert_type(t_vec, jnp.int32)
        t_uk = _to_ukey(t_bits)

        target = jnp.int32(M - 1)
        pref = jnp.int32(0)
        for d in range(4):
            shift = 24 - 8 * d
            for kk in range(16):
                hist_v[pl.ds(kk * L, L)] = zero_i
            pref_vec = jnp.full((L,), pref)

            def h_body(i, carry, shift=shift, d=d, pref_vec=pref_vec):
                for u in range(UNR):
                    uk = keys_v[pl.ds(i * (UNR * L) + u * L, L)]
                    idx = lax.shift_right_logical(uk, shift) & 0xFF
                    if d == 0:
                        plsc.addupdate_scatter(hist_v, [idx], one_i)
                    else:
                        match = (lax.shift_right_logical(uk, shift + 8)
                                 == pref_vec)
                        plsc.addupdate_scatter(hist_v, [idx], one_i,
                                               mask=match)
                return carry

            lax.fori_loop(0, CHUNKS // UNR, h_body, jnp.int32(0))

            t_idx = lax.shift_right_logical(t_uk, shift) & 0xFF
            if d == 0:
                tmask = lane == 0
            else:
                tmask = (lane == 0) & (
                    lax.shift_right_logical(t_uk, shift + 8) == pref_vec)
            plsc.addupdate_scatter(hist_v, [t_idx], -one_i, mask=tmask)

            hs = [hist_v[pl.ds(kk * L, L)] for kk in range(16)]
            ssum = [jnp.sum(h) for h in hs]
            s_ge = jnp.int32(0)
            suffix = [None] * 16
            for kk in reversed(range(16)):
                suffix[kk] = s_ge + ssum[kk]
                s_ge = suffix[kk]
            tgt_vec = jnp.full((L,), target)
            cnt_true = jnp.int32(0)
            for kk in range(16):
                csum = plsc.cumsum(hs[kk])
                tvals = jnp.full((L,), suffix[kk]) - csum + hs[kk]
                cnt_true = cnt_true + jnp.max(
                    plsc.all_reduce_population_count(tvals >= tgt_vec))
            b_star = cnt_true - 1
            b_vec = jnp.full((L,), b_star)
            d_b = jnp.max(plsc.load_gather(hist_v, [b_vec]))
            cnt_ge = jnp.int32(0)
            for kk in range(16):
                cnt_ge = cnt_ge + jnp.sum(
                    jnp.where(lane + (kk * L) >= b_vec, hs[kk], zero_i))
            target = target - (cnt_ge - d_b)
            pref = lax.shift_left(pref, 8) | b_star

        theta_u = jnp.full((L,), pref)
        theta_bits = jnp.where(theta_u < 0, theta_u ^ MININT, ~theta_u)
        theta_vec = lax.bitcast_convert_type(theta_bits, jnp.float32)

        w1 = levels[0]

        def fin_body(i, carry):
            sexp, z0, z1, z2, z3, z4, cnt = carry
            for u in range(UNR):
                chunk = row_v[pl.ds(i * (UNR * L) + u * L, L)]
                gt = chunk > theta_vec
                cnt = cnt + jnp.where(gt, one_i, zero_i)
                sexp = sexp + jnp.where(gt, jnp.exp(chunk - w1), 0.0)
                z0 = z0 + jnp.where(
                    gt, jnp.exp(-jnp.abs(chunk - levels[0]) * INV_TAU), 0.0)
                z1 = z1 + jnp.where(
                    gt, jnp.exp(-jnp.abs(chunk - levels[1]) * INV_TAU), 0.0)
                z2 = z2 + jnp.where(
                    gt, jnp.exp(-jnp.abs(chunk - levels[2]) * INV_TAU), 0.0)
                z3 = z3 + jnp.where(
                    gt, jnp.exp(-jnp.abs(chunk - levels[3]) * INV_TAU), 0.0)
                z4 = z4 + jnp.where(
                    gt, jnp.exp(-jnp.abs(chunk - levels[4]) * INV_TAU), 0.0)
            return sexp, z0, z1, z2, z3, z4, cnt

        sexp, z0, z1, z2, z3, z4, cnt = lax.fori_loop(
            0, CHUNKS // UNR, fin_body,
            (zero_f, zero_f, zero_f, zero_f, zero_f, zero_f, zero_i))
        zsums = [jnp.full((L,), jnp.sum(z)) for z in (z0, z1, z2, z3, z4)]
        sexp_v = jnp.full((L,), jnp.sum(sexp))
        cnt_v = jnp.full((L,), jnp.sum(cnt))

        tgt = t_vec > theta_vec
        cnt_false = cnt_v - jnp.where(tgt, one_i, zero_i)
        r_f = (jnp.full((L,), jnp.int32(M - 1)) - cnt_false).astype(
            jnp.float32)
        et = jnp.exp(t_vec - w1)
        sum_exp = (sexp_v + jnp.where(tgt, zero_f, et)
                   + r_f * jnp.exp(theta_vec - w1))
        pv = 0.2 * et / sum_exp

        cum = jnp.int32(0)
        for p in range(K):
            a_v = jnp.full((L,), cum)
            cum = cum + counts[p]
            b_v = jnp.full((L,), cum)
            cmask = (lane >= a_v) & (lane < b_v) & (lane < K)
            coeff_lane = 0.2 * jnp.minimum(float(K) - lane_f, 4.0)
            cs_v = jnp.full((L,), jnp.sum(jnp.where(cmask, coeff_lane, 0.0)))
            numer = jnp.exp(-jnp.abs(t_vec - levels[p]) * INV_TAU)
            z_tot = (zsums[p] + jnp.where(tgt, zero_f, numer)
                     + r_f * jnp.exp(-jnp.abs(theta_vec - levels[p])
                                     * INV_TAU))
            pv = pv + cs_v * numer / jnp.maximum(z_tot, 1e-30)

        pvec = jnp.where(lane == j, pv, pvec)

    stage_v[...] = pvec
    pltpu.sync_copy(stage_v, out_hbm.at[wid])


def kernel(outputs, labels):
    p2d = _sc_loss_kernel(outputs, labels)
    p = p2d[:, :RPW].reshape(B)
    return jnp.mean(-jnp.log(p * (1.0 - 2e-07) + 1e-07))


# speedup vs baseline: 10.0079x; 1.1378x over previous
import functools

import jax
import jax.numpy as jnp
from jax import lax
from jax.experimental import pallas as pl
from jax.experimental.pallas import tpu as pltpu
from jax.experimental.pallas import tpu_sc as plsc

B = 128
N = 8192
M = 512
K = 5
INV_TAU = 16.0
NC = 2
NS = 16
L = 16
NW = NC * NS
RPW = B // NW
CHUNKS = N // L
UNR = 4
NEG = float("-inf")
MININT = -2147483648

_mesh = plsc.VectorSubcoreMesh(core_axis_name="c", subcore_axis_name="s")


def _to_ukey(bits):
    return bits ^ (lax.shift_right_arithmetic(bits, 31) | MININT)


@functools.partial(
    pl.kernel,
    out_type=jax.ShapeDtypeStruct((NW, L), jnp.float32),
    mesh=_mesh,
    compiler_params=pltpu.CompilerParams(needs_layout_passes=False),
    scratch_types=[
        pltpu.VMEM((N,), jnp.float32),
        pltpu.VMEM((N,), jnp.float32),
        pltpu.VMEM((N,), jnp.int32),
        pltpu.VMEM((B,), jnp.int32),
        pltpu.VMEM((256,), jnp.int32),
        pltpu.VMEM((L,), jnp.float32),
        pltpu.SemaphoreType.DMA,
        pltpu.SemaphoreType.DMA,
    ],
)
def _sc_loss_kernel(outputs_hbm, labels_hbm, out_hbm, row_a, row_b, keys_v,
                    labels_v, hist_v, stage_v, sem_a, sem_b):
    wid = lax.axis_index("s") * NC + lax.axis_index("c")
    pltpu.sync_copy(labels_hbm, labels_v)
    lane = lax.iota(jnp.int32, L)
    lane_f = lane.astype(jnp.float32)
    zero_f = jnp.zeros((L,), jnp.float32)
    zero_i = jnp.zeros((L,), jnp.int32)
    one_i = jnp.ones((L,), jnp.int32)
    neg_v = jnp.full((L,), NEG, jnp.float32)

    bufs = [(row_a, sem_a), (row_b, sem_b)]
    row0 = wid * RPW
    pend = pltpu.async_copy(outputs_hbm.at[row0], row_a, sem_a)

    pvec = zero_f
    for j in range(RPW):
        r = row0 + j
        row_v = bufs[j % 2][0]
        pend.wait()
        if j + 1 < RPW:
            nbuf, nsem = bufs[(j + 1) % 2]
            pend = pltpu.async_copy(outputs_hbm.at[r + 1], nbuf, nsem)
        lab_vec = plsc.load_gather(labels_v, [jnp.full((L,), r, jnp.int32)])
        t_vec = plsc.load_gather(row_v, [lab_vec])

        for kk in range(16):
            hist_v[pl.ds(kk * L, L)] = zero_i

        def p1_body(i, carry, row_v=row_v):
            s1, s2, s3, s4, s5 = carry
            for u in range(UNR):
                chunk = row_v[pl.ds(i * (UNR * L) + u * L, L)]
                bits = lax.bitcast_convert_type(chunk, jnp.int32)
                uk = _to_ukey(bits)
                keys_v[pl.ds(i * (UNR * L) + u * L, L)] = uk
                plsc.addupdate_scatter(
                    hist_v, [lax.shift_right_logical(uk, 24)], one_i)
                x = chunk
                n1 = jnp.maximum(s1, x)
                x = jnp.minimum(s1, x)
                n2 = jnp.maximum(s2, x)
                x = jnp.minimum(s2, x)
                n3 = jnp.maximum(s3, x)
                x = jnp.minimum(s3, x)
                n4 = jnp.maximum(s4, x)
                x = jnp.minimum(s4, x)
                n5 = jnp.maximum(s5, x)
                s1, s2, s3, s4, s5 = n1, n2, n3, n4, n5
            return s1, s2, s3, s4, s5

        svecs = list(lax.fori_loop(0, CHUNKS // UNR, p1_body, (neg_v,) * 5))

        levels = [jnp.full((L,), jnp.max(svecs[0]))]
        counts = []
        for _p in range(K - 1):
            prev = levels[-1]
            macc, cacc = neg_v, zero_i
            for s in svecs:
                macc = jnp.maximum(macc, jnp.where(s < prev, s, NEG))
                cacc = cacc + jnp.where(s == prev, one_i, zero_i)
            counts.append(jnp.sum(cacc))
            levels.append(jnp.full((L,), jnp.max(macc)))
        cacc = zero_i
        for s in svecs:
            cacc = cacc + jnp.where(s == levels[-1], one_i, zero_i)
        counts.append(jnp.sum(cacc))

        cums = []
        cum = jnp.int32(0)
        for p in range(K):
            cum = cum + counts[p]
            cums.append(cum)
        w5 = levels[K - 1]
        for p in reversed(range(K - 1)):
            w5 = jnp.where(jnp.full((L,), cums[p]) >= 5, levels[p], w5)

        t_bits = lax.bitcast_convert_type(t_vec, jnp.int32)
        t_uk = _to_ukey(t_bits)

        target = jnp.int32(M - 1)
        pref = jnp.int32(0)
        for d in range(4):
            shift = 24 - 8 * d
            pref_vec = jnp.full((L,), pref)
            if d > 0:
                for kk in range(16):
                    hist_v[pl.ds(kk * L, L)] = zero_i

                def h_body(i, carry, shift=shift, pref_vec=pref_vec):
                    for u in range(UNR):
                        uk = keys_v[pl.ds(i * (UNR * L) + u * L, L)]
                        idx = lax.shift_right_logical(uk, shift) & 0xFF
                        match = (lax.shift_right_logical(uk, shift + 8)
                                 == pref_vec)
                        plsc.addupdate_scatter(hist_v, [idx], one_i,
                                               mask=match)
                    return carry

                lax.fori_loop(0, CHUNKS // UNR, h_body, jnp.int32(0))

            t_idx = lax.shift_right_logical(t_uk, shift) & 0xFF
            if d == 0:
                tmask = lane == 0
            else:
                tmask = (lane == 0) & (
                    lax.shift_right_logical(t_uk, shift + 8) == pref_vec)
            plsc.addupdate_scatter(hist_v, [t_idx], -one_i, mask=tmask)

            hs = [hist_v[pl.ds(kk * L, L)] for kk in range(16)]
            ssum = [jnp.sum(h) for h in hs]
            s_ge = jnp.int32(0)
            suffix = [None] * 16
            for kk in reversed(range(16)):
                suffix[kk] = s_ge + ssum[kk]
                s_ge = suffix[kk]
            tgt_vec = jnp.full((L,), target)
            cnt_true = jnp.int32(0)
            for kk in range(16):
                csum = plsc.cumsum(hs[kk])
                tvals = jnp.full((L,), suffix[kk]) - csum + hs[kk]
                cnt_true = cnt_true + jnp.max(
                    plsc.all_reduce_population_count(tvals >= tgt_vec))
            b_star = cnt_true - 1
            b_vec = jnp.full((L,), b_star)
            d_b = jnp.max(plsc.load_gather(hist_v, [b_vec]))
            cnt_ge = jnp.int32(0)
            for kk in range(16):
                cnt_ge = cnt_ge + jnp.sum(
                    jnp.where(lane + (kk * L) >= b_vec, hs[kk], zero_i))
            target = target - (cnt_ge - d_b)
            pref = lax.shift_left(pref, 8) | b_star

        theta_u = jnp.full((L,), pref)
        theta_bits = jnp.where(theta_u < 0, theta_u ^ MININT, ~theta_u)
        theta_vec = lax.bitcast_convert_type(theta_bits, jnp.float32)

        w1 = levels[0]

        def fin_body(i, carry, row_v=row_v):
            sexp, ee, n5c, cnt = carry
            for u in range(UNR):
                chunk = row_v[pl.ds(i * (UNR * L) + u * L, L)]
                gt = chunk > theta_vec
                cnt = cnt + jnp.where(gt, one_i, zero_i)
                sexp = sexp + jnp.where(gt, jnp.exp(chunk - w1), 0.0)
                ee = ee + jnp.where(
                    gt & (chunk < w5),
                    jnp.exp((chunk - w5) * INV_TAU), 0.0)
                n5c = n5c + jnp.where(gt & (chunk == w5), one_i, zero_i)
            return sexp, ee, n5c, cnt

        sexp, ee, n5c, cnt = lax.fori_loop(
            0, CHUNKS // UNR, fin_body, (zero_f, zero_f, zero_i, zero_i))
        sexp_v = jnp.full((L,), jnp.sum(sexp))
        e_v = jnp.full((L,), jnp.sum(ee))
        n5_v = jnp.full((L,), jnp.sum(n5c)).astype(jnp.float32)
        cnt_v = jnp.full((L,), jnp.sum(cnt))

        tgt = t_vec > theta_vec
        cnt_false = cnt_v - jnp.where(tgt, one_i, zero_i)
        r_f = (jnp.full((L,), jnp.int32(M - 1)) - cnt_false).astype(
            jnp.float32)
        et = jnp.exp(t_vec - w1)
        sum_exp = (sexp_v + jnp.where(tgt, zero_f, et)
                   + r_f * jnp.exp(theta_vec - w1))
        pv = 0.2 * et / sum_exp

        base = e_v + n5_v
        cq_f = [counts[q].astype(jnp.float32) for q in range(K - 1)]
        cum = jnp.int32(0)
        for p in range(K):
            a_v = jnp.full((L,), cum)
            cum = cum + counts[p]
            b_v = jnp.full((L,), cum)
            cmask = (lane >= a_v) & (lane < b_v) & (lane < K)
            coeff_lane = 0.2 * jnp.minimum(float(K) - lane_f, 4.0)
            cs_v = jnp.full((L,), jnp.sum(jnp.where(cmask, coeff_lane, 0.0)))
            numer = jnp.exp(-jnp.abs(t_vec - levels[p]) * INV_TAU)
            z_tot = base * jnp.exp(
                -jnp.maximum(levels[p] - w5, 0.0) * INV_TAU)
            for q in range(K - 1):
                z_tot = z_tot + jnp.where(
                    levels[q] > w5,
                    cq_f[q] * jnp.exp(-jnp.abs(levels[q] - levels[p])
                                      * INV_TAU),
                    zero_f)
            z_tot = z_tot + r_f * jnp.exp(
                -jnp.maximum(levels[p] - theta_vec, 0.0) * INV_TAU)
            z_tot = z_tot + jnp.where(tgt, zero_f, numer)
            pv = pv + cs_v * numer / jnp.maximum(z_tot, 1e-30)

        pvec = jnp.where(lane == j, pv, pvec)

    stage_v[...] = pvec
    pltpu.sync_copy(stage_v, out_hbm.at[wid])


def kernel(outputs, labels):
    p2d = _sc_loss_kernel(outputs, labels)
    p = p2d[:, :RPW].reshape(B)
    return jnp.mean(-jnp.log(p * (1.0 - 2e-07) + 1e-07))
